# baseline (speedup 1.0000x reference)
import jax
import jax.numpy as jnp
from jax import lax
from jax.experimental import pallas as pl
from jax.experimental.pallas import tpu as pltpu
from jax.experimental.pallas import tpu_sc as plsc

_B, _C, _D = 1024, 32, 64
_NW = 32
_BPW = _B // _NW


def _sc_reduce_body(x_hbm, o_hbm, rows_v, out_v):
    wid = lax.axis_index("s") * 2 + lax.axis_index("c")
    base = wid * _BPW
    pltpu.sync_copy(x_hbm.at[pl.ds(base, _BPW)], rows_v)

    def per_graph(g, carry):
        for l in range(_D // 16):
            acc = rows_v[g, pl.ds(l * 16, 16)]
            for c in range(1, _C):
                acc = acc + rows_v[g, pl.ds(c * _D + l * 16, 16)]
            out_v[g, pl.ds(l * 16, 16)] = acc * (1.0 / _C)
        return carry

    lax.fori_loop(0, _BPW, per_graph, 0)
    pltpu.sync_copy(out_v, o_hbm.at[pl.ds(base, _BPW)])


def _channel_means(flat):
    mesh = plsc.VectorSubcoreMesh(core_axis_name="c", subcore_axis_name="s")
    return pl.kernel(
        _sc_reduce_body,
        out_type=jax.ShapeDtypeStruct((_B, _D), jnp.float32),
        mesh=mesh,
        scratch_types=[
            pltpu.VMEM((_BPW, _C * _D), jnp.float32),
            pltpu.VMEM((_BPW, _D), jnp.float32),
        ],
    )(flat)


def _mlp_body(m_ref, w1_ref, b1_ref, w2_ref, b2_ref, w3_ref, b3_ref,
              wr_ref, br_ref, o_ref):
    m = m_ref[...]
    h = jnp.maximum(
        jnp.dot(m, w1_ref[...], preferred_element_type=jnp.float32)
        + b1_ref[...], 0.0)
    h = jnp.maximum(
        jnp.dot(h, w2_ref[...], preferred_element_type=jnp.float32)
        + b2_ref[...], 0.0)
    h = jnp.maximum(
        jnp.dot(h, w3_ref[...], preferred_element_type=jnp.float32)
        + b3_ref[...], 0.0)
    o_ref[...] = (
        jnp.dot(h, wr_ref[...], preferred_element_type=jnp.float32)
        + br_ref[...])


def kernel(embeddings, W1, b1, W2, b2, W3, b3, Wr, br):
    B, C, D = embeddings.shape
    flat = embeddings.reshape(B, C * D)
    m = _channel_means(flat)
    return pl.pallas_call(
        _mlp_body,
        out_shape=jax.ShapeDtypeStruct((B, D), jnp.float32),
    )(m, W1, b1, W2, b2, W3, b3, Wr, br)


# speedup vs baseline: 2.4779x; 2.4779x over previous
import jax
import jax.numpy as jnp
from jax.experimental import pallas as pl

B_BLOCK = 512


def _mlp_kernel(x_ref, w1_ref, b1_ref, w2_ref, b2_ref, w3_ref, b3_ref,
                wr_ref, br_ref, o_ref):
    x = x_ref[...]
    w = x.shape[1]
    while w > 64:
        w //= 2
        x = x[:, :w] + x[:, w:]
    m = x * (1.0 / 32.0)
    h = jnp.maximum(
        jnp.dot(m, w1_ref[...], preferred_element_type=jnp.float32)
        + b1_ref[...], 0.0)
    h = jnp.maximum(
        jnp.dot(h, w2_ref[...], preferred_element_type=jnp.float32)
        + b2_ref[...], 0.0)
    h = jnp.maximum(
        jnp.dot(h, w3_ref[...], preferred_element_type=jnp.float32)
        + b3_ref[...], 0.0)
    o_ref[...] = (
        jnp.dot(h, wr_ref[...], preferred_element_type=jnp.float32)
        + br_ref[...])


def kernel(embeddings, W1, b1, W2, b2, W3, b3, Wr, br):
    B, C, D = embeddings.shape
    H = W1.shape[1]
    grid = (B // B_BLOCK,)
    flat = embeddings.reshape(B, C * D)

    def full(shape):
        return pl.BlockSpec(shape, lambda i: (0,) * len(shape))

    return pl.pallas_call(
        _mlp_kernel,
        grid=grid,
        in_specs=[
            pl.BlockSpec((B_BLOCK, C * D), lambda i: (i, 0)),
            full((D, H)), full((H,)),
            full((H, H)), full((H,)),
            full((H, H)), full((H,)),
            full((H, D)), full((D,)),
        ],
        out_specs=pl.BlockSpec((B_BLOCK, D), lambda i: (i, 0)),
        out_shape=jax.ShapeDtypeStruct((B, D), jnp.float32),
    )(flat, W1, b1, W2, b2, W3, b3, Wr, br)
